# Initial kernel scaffold; baseline (speedup 1.0000x reference)
import jax
import jax.numpy as jnp
from jax.experimental import pallas as pl


def kernel(hidden_states, gate_w, gate_proj, up_proj, down_proj):
    raise NotImplementedError("write your pallas kernel here")


# speedup vs baseline: 1.0020x; 1.0020x over previous
import functools

import jax
import jax.numpy as jnp
from jax import lax
from jax.experimental import pallas as pl
from jax.experimental.pallas import tpu as pltpu
from jax.experimental.pallas import tpu_sc as plsc

T, H, E, K, FF = 2048, 1024, 8, 2, 2816
M = 256
PADDED = T * K + E * M
NB = PADDED // M
FF_BLK = 256
NJ = FF // FF_BLK
EPAD = 128

NC, NS = 2, 16
NW = NC * NS
G_ROWS = PADDED // NW
G_CH = 32
G_NCH = G_ROWS // G_CH
C_CH = 16
C_NCH = (T // NW) // C_CH


def _router_body(x_ref, gw_ref, e0_ref, e1_ref, w0_ref):
    logits = lax.dot_general(x_ref[...], gw_ref[...],
                             (((1,), (1,)), ((), ())),
                             preferred_element_type=jnp.float32)
    rows = logits.shape[0]
    iota = lax.broadcasted_iota(jnp.int32, (rows, EPAD), 1)
    masked = jnp.where(iota < E, logits, -1e30)
    m0 = jnp.max(masked, axis=1, keepdims=True)
    e0 = jnp.min(jnp.where(masked == m0, iota, EPAD), axis=1, keepdims=True)
    l2 = jnp.where(iota == e0, -1e30, masked)
    m1 = jnp.max(l2, axis=1, keepdims=True)
    e1 = jnp.min(jnp.where(l2 == m1, iota, EPAD), axis=1, keepdims=True)
    w0 = jax.nn.sigmoid(m0 - m1)
    e0_ref[...] = jnp.broadcast_to(e0, (rows, EPAD))
    e1_ref[...] = jnp.broadcast_to(e1, (rows, EPAD))
    w0_ref[...] = jnp.broadcast_to(w0, (rows, EPAD))


def _router(x, gwp):
    rb = 512
    return pl.pallas_call(
        _router_body,
        grid=(T // rb,),
        in_specs=[pl.BlockSpec((rb, H), lambda i: (i, 0)),
                  pl.BlockSpec((EPAD, H), lambda i: (0, 0))],
        out_specs=[pl.BlockSpec((rb, EPAD), lambda i: (i, 0)),
                   pl.BlockSpec((rb, EPAD), lambda i: (i, 0)),
                   pl.BlockSpec((rb, EPAD), lambda i: (i, 0))],
        out_shape=[jax.ShapeDtypeStruct((T, EPAD), jnp.int32),
                   jax.ShapeDtypeStruct((T, EPAD), jnp.int32),
                   jax.ShapeDtypeStruct((T, EPAD), jnp.float32)],
    )(x, gwp)


def _gather_body(x_hbm, idx_hbm, out_hbm, idx_v, rows_v, sem):
    wid = lax.axis_index("s") * NC + lax.axis_index("c")
    pltpu.sync_copy(idx_hbm.at[wid], idx_v)
    base = wid * G_ROWS
    for c in range(G_NCH):
        pltpu.async_copy(x_hbm.at[idx_v.at[c]], rows_v, sem).wait()
        pltpu.sync_copy(rows_v, out_hbm.at[pl.ds(base + c * G_CH, G_CH)])


def _gather(x, idx2d):
    mesh = plsc.VectorSubcoreMesh(core_axis_name="c", subcore_axis_name="s",
                                  num_cores=NC)
    return pl.kernel(
        _gather_body,
        out_type=jax.ShapeDtypeStruct((PADDED, H), jnp.float32),
        mesh=mesh,
        scratch_types=[pltpu.VMEM((G_NCH, G_CH), jnp.int32),
                       pltpu.VMEM((G_CH, H), jnp.float32),
                       pltpu.SemaphoreType.DMA],
    )(x, idx2d)


def _ffn_body(be_ref, xs_ref, wg_ref, wu_ref, wd_ref, wrep_ref, out_ref):
    j = pl.program_id(1)

    @pl.when(j == 0)
    def _():
        out_ref[...] = jnp.zeros_like(out_ref)

    xb = xs_ref[...].astype(jnp.bfloat16)
    wg = wg_ref[0].astype(jnp.bfloat16)
    wu = wu_ref[0].astype(jnp.bfloat16)
    wd = wd_ref[0].astype(jnp.bfloat16)
    nt = (((1,), (1,)), ((), ()))
    g = lax.dot_general(xb, wg, nt, preferred_element_type=jnp.float32)
    u = lax.dot_general(xb, wu, nt, preferred_element_type=jnp.float32)
    act = (jax.nn.silu(g) * u).astype(jnp.bfloat16)
    y = lax.dot_general(act, wd, nt, preferred_element_type=jnp.float32)
    out_ref[...] += y

    @pl.when(j == NJ - 1)
    def _():
        out_ref[...] = out_ref[...] * wrep_ref[:, 0:1]


def _ffn(block_expert, xs, gate_proj, up_proj, down_proj, w_rep):
    grid_spec = pltpu.PrefetchScalarGridSpec(
        num_scalar_prefetch=1,
        grid=(NB, NJ),
        in_specs=[
            pl.BlockSpec((M, H), lambda i, j, be: (i, 0)),
            pl.BlockSpec((1, FF_BLK, H), lambda i, j, be: (be[i], j, 0)),
            pl.BlockSpec((1, FF_BLK, H), lambda i, j, be: (be[i], j, 0)),
            pl.BlockSpec((1, H, FF_BLK), lambda i, j, be: (be[i], 0, j)),
            pl.BlockSpec((M, 128), lambda i, j, be: (i, 0)),
        ],
        out_specs=pl.BlockSpec((M, H), lambda i, j, be: (i, 0)),
    )
    return pl.pallas_call(
        _ffn_body,
        grid_spec=grid_spec,
        out_shape=jax.ShapeDtypeStruct((PADDED, H), jnp.float32),
        compiler_params=pltpu.CompilerParams(
            dimension_semantics=("arbitrary", "arbitrary")),
    )(block_expert, xs, gate_proj, up_proj, down_proj, w_rep)


def _combine_body(ys_hbm, p0_hbm, p1_hbm, out_hbm, p0_v, p1_v, a_v, b_v, sem):
    wid = lax.axis_index("s") * NC + lax.axis_index("c")
    pltpu.sync_copy(p0_hbm.at[wid], p0_v)
    pltpu.sync_copy(p1_hbm.at[wid], p1_v)
    base = wid * C_NCH * C_CH
    for c in range(C_NCH):
        pltpu.async_copy(ys_hbm.at[p0_v.at[c]], a_v, sem).wait()
        pltpu.async_copy(ys_hbm.at[p1_v.at[c]], b_v, sem).wait()
        for r in range(C_CH):
            def add_body(i, _, r=r):
                sl = pl.ds(i * 16, 16)
                a_v[r, sl] += b_v[r, sl]
                return 0
            lax.fori_loop(0, H // 16, add_body, 0)
        pltpu.sync_copy(a_v, out_hbm.at[pl.ds(base + c * C_CH, C_CH)])


def _combine(ys, p0_2d, p1_2d):
    mesh = plsc.VectorSubcoreMesh(core_axis_name="c", subcore_axis_name="s",
                                  num_cores=NC)
    return pl.kernel(
        _combine_body,
        out_type=jax.ShapeDtypeStruct((T, H), jnp.float32),
        mesh=mesh,
        scratch_types=[pltpu.VMEM((C_NCH, C_CH), jnp.int32),
                       pltpu.VMEM((C_NCH, C_CH), jnp.int32),
                       pltpu.VMEM((C_CH, H), jnp.float32),
                       pltpu.VMEM((C_CH, H), jnp.float32),
                       pltpu.SemaphoreType.DMA],
    )(ys, p0_2d, p1_2d)


def kernel(hidden_states, gate_w, gate_proj, up_proj, down_proj):
    b, s, h = hidden_states.shape
    assert (b * s, h) == (T, H) and gate_w.shape == (E, H)
    x = hidden_states.reshape(T, H)
    gwp = jnp.zeros((EPAD, H), jnp.float32).at[:E].set(gate_w)

    e0b, e1b, w0b = _router(x, gwp)
    e0, e1, w0 = e0b[:, 0], e1b[:, 0], w0b[:, 0]
    w1 = 1.0 - w0

    flat_e = jnp.stack([e0, e1], axis=1).reshape(-1)
    flat_w = jnp.stack([w0, w1], axis=1).reshape(-1)
    onehot = (flat_e[:, None] == jnp.arange(E)[None, :]).astype(jnp.int32)
    counts = jnp.sum(onehot, axis=0)
    rank = jnp.sum((jnp.cumsum(onehot, axis=0) - onehot) * onehot, axis=1)
    padded_counts = ((counts + M - 1) // M) * M
    starts = jnp.cumsum(padded_counts) - padded_counts
    dst = starts[flat_e] + rank
    src_token = jnp.zeros((PADDED,), jnp.int32).at[dst].set(
        (jnp.arange(T * K) // K).astype(jnp.int32))
    w_sorted = jnp.zeros((PADDED,), jnp.float32).at[dst].set(flat_w)
    block_expert = (jnp.searchsorted(starts // M, jnp.arange(NB), side="right")
                    - 1).astype(jnp.int32)
    block_expert = jnp.clip(block_expert, 0, E - 1)
    pos = dst.reshape(T, K).astype(jnp.int32)

    xs = _gather(x, src_token.reshape(NW, G_NCH, G_CH))
    w_rep = jnp.broadcast_to(w_sorted[:, None], (PADDED, 128))
    ys = _ffn(block_expert, xs, gate_proj, up_proj, down_proj, w_rep)
    out = _combine(ys,
                   pos[:, 0].reshape(NW, C_NCH, C_CH),
                   pos[:, 1].reshape(NW, C_NCH, C_CH))
    return out.reshape(b, s, h)
